# Initial kernel scaffold; baseline (speedup 1.0000x reference)
import jax
import jax.numpy as jnp
from jax.experimental import pallas as pl


def kernel(x_drug, x_protein, edge_src, edge_dst, neg_src, neg_dst, w_sd1, w_sp1, w_dp1, w_pd1, w_sd2, w_sp2, w_dp2, w_pd2, w_pred, b_pred):
    raise NotImplementedError("write your pallas kernel here")


# speedup vs baseline: 7.6437x; 7.6437x over previous
import functools

import jax
import jax.numpy as jnp
from jax import lax
from jax.experimental import pallas as pl
from jax.experimental.pallas import tpu as pltpu
from jax.experimental.pallas import tpu_sc as plsc

ND = 5000
NP = 5000
E = 160000
F = 128

NPAD = 5120
EPAD = 163840
CH = 64
NC, NS = 2, 16
NW = NC * NS
CPT = EPAD // CH // NS
NB = 2
IDXB = 32
RPT = NPAD // NS
EPW = EPAD // NW
BLK = 512

_mesh = plsc.VectorSubcoreMesh(
    core_axis_name="c", subcore_axis_name="s", num_cores=NC, num_subcores=NS)

_f32 = jnp.float32


NROWS = NPAD // 128


@functools.partial(
    pl.kernel,
    out_type=(
        jax.ShapeDtypeStruct((NPAD, F), _f32),
        jax.ShapeDtypeStruct((NPAD, F), _f32),
        jax.ShapeDtypeStruct((NROWS, 128), _f32),
        jax.ShapeDtypeStruct((NROWS, 128), _f32),
    ),
    mesh=_mesh,
    scratch_types=[
        pltpu.VMEM_SHARED((NPAD, F), _f32),
        pltpu.VMEM_SHARED((NPAD, F), _f32),
        pltpu.VMEM_SHARED((NROWS, 128), _f32),
        pltpu.VMEM((IDXB, CH), jnp.int32),
        pltpu.VMEM((IDXB, CH), jnp.int32),
        pltpu.VMEM((NROWS, 128), _f32),
        pltpu.VMEM((NROWS,), jnp.int32),
        pltpu.VMEM((CH, F), _f32),
        pltpu.VMEM((CH, F), _f32),
        pltpu.SemaphoreType.DMA,
        pltpu.SemaphoreType.DMA,
        pltpu.SemaphoreType.DMA,
        pltpu.SemaphoreType.DMA,
    ],
    compiler_params=pltpu.CompilerParams(needs_layout_passes=False),
)
def _sc_aggregate(xd_hbm, xp_hbm, src3_hbm, dst3_hbm, zrow_hbm, iota_hbm,
                  aggp_out, aggd_out, cntp_out, cntd_out,
                  acc, x_sh, cnt_sh, by_idx, at_idx, hist, iota_v,
                  rows0, rows1, gsem0, gsem1, ssem0, ssem1):
    c = lax.axis_index("c")
    s = lax.axis_index("s")
    r0 = s * RPT

    pltpu.sync_copy(zrow_hbm.at[pl.ds(r0, RPT)], acc.at[pl.ds(r0, RPT)])
    pltpu.sync_copy(zrow_hbm.at[pl.ds(0, NROWS)], hist)
    pltpu.sync_copy(iota_hbm, iota_v)

    @pl.when(s == 0)
    def _():
        pltpu.sync_copy(zrow_hbm.at[pl.ds(0, NROWS)], cnt_sh)

    rows = (rows0, rows1)
    gsems = (gsem0, gsem1)
    ssems = (ssem0, ssem1)
    ones16 = jnp.ones((16,), _f32)

    def run(x_hbm, by3_hbm, at3_hbm, out_hbm, cnt_out):
        pltpu.sync_copy(x_hbm.at[pl.ds(r0, RPT)], x_sh.at[pl.ds(r0, RPT)])
        plsc.subcore_barrier()

        @pl.loop(0, CPT // IDXB)
        def _blk(q):
            pltpu.sync_copy(by3_hbm.at[s, pl.ds(q * IDXB, IDXB)], by_idx)
            pltpu.sync_copy(at3_hbm.at[s, pl.ds(q * IDXB, IDXB)], at_idx)
            for b in range(NB):
                pltpu.async_copy(x_sh.at[by_idx.at[b]], rows[b], gsems[b])

            @pl.loop(0, IDXB // NB)
            def _grp(g):
                for b in range(NB):
                    i = g * NB + b
                    pltpu.make_async_copy(x_sh.at[by_idx.at[i]], rows[b], gsems[b]).wait()
                    pltpu.async_copy(rows[b], acc.at[at_idx.at[i]], ssems[b], add=True)

                    @pl.loop(0, CH // 16)
                    def _h(j):
                        i16 = at_idx[i, pl.ds(j * 16, 16)]
                        plsc.addupdate_scatter(hist, [i16 >> 7, i16 & 127], ones16)

                @pl.when(g < IDXB // NB - 1)
                def _():
                    for b in range(NB):
                        i = g * NB + b
                        pltpu.make_async_copy(rows[b], acc.at[at_idx.at[i]], ssems[b]).wait()
                        pltpu.async_copy(x_sh.at[by_idx.at[(g + 1) * NB + b]], rows[b], gsems[b])

            last = IDXB - NB
            for b in range(NB):
                pltpu.make_async_copy(rows[b], acc.at[at_idx.at[last + b]], ssems[b]).wait()

        pltpu.sync_copy(hist, cnt_sh.at[iota_v], add=True)
        plsc.subcore_barrier()
        pltpu.sync_copy(acc.at[pl.ds(r0, RPT)], out_hbm.at[pl.ds(r0, RPT)])

        @pl.when(s == 0)
        def _():
            pltpu.sync_copy(cnt_sh, cnt_out)

    @pl.when(c == 0)
    def _():
        run(xd_hbm, src3_hbm, dst3_hbm, aggp_out, cntp_out)

    @pl.when(c == 1)
    def _():
        run(xp_hbm, dst3_hbm, src3_hbm, aggd_out, cntd_out)


_row = lambda e, r: (e, r, 0)
_wmap = lambda e, r: (e, 0, 0)


def _tc_layer1_body(x_ref, p_ref, c_ref, ws_ref, wc_ref,
                    h_ref, dinv_ref):
    dinv = 1.0 / jnp.maximum(c_ref[0], 1.0)
    agg = p_ref[0] * dinv
    h = jnp.dot(x_ref[0], ws_ref[0], preferred_element_type=_f32)
    h = h + jnp.dot(agg, wc_ref[0], preferred_element_type=_f32)
    h_ref[0] = jnp.maximum(h, 0.0)
    dinv_ref[0] = dinv


_tc_layer1 = pl.pallas_call(
    _tc_layer1_body,
    grid=(2, NPAD // BLK),
    in_specs=[
        pl.BlockSpec((1, BLK, F), _row),
        pl.BlockSpec((1, BLK, F), _row),
        pl.BlockSpec((1, BLK, 1), _row),
        pl.BlockSpec((1, F, F), _wmap),
        pl.BlockSpec((1, F, F), _wmap),
    ],
    out_specs=[
        pl.BlockSpec((1, BLK, F), _row),
        pl.BlockSpec((1, BLK, 1), _row),
    ],
    out_shape=[
        jax.ShapeDtypeStruct((2, NPAD, F), _f32),
        jax.ShapeDtypeStruct((2, NPAD, 1), _f32),
    ],
)


def _tc_layer2_body(x_ref, p_ref, dinv_ref, ws_ref, wc_ref, wp_ref,
                    h_ref, a_ref):
    agg = p_ref[0] * dinv_ref[0]
    h = jnp.dot(x_ref[0], ws_ref[0], preferred_element_type=_f32)
    h = h + jnp.dot(agg, wc_ref[0], preferred_element_type=_f32)
    h_ref[0] = h
    a_ref[0] = jnp.sum(h * wp_ref[0], axis=1, keepdims=True)


_tc_layer2 = pl.pallas_call(
    _tc_layer2_body,
    grid=(2, NPAD // BLK),
    in_specs=[
        pl.BlockSpec((1, BLK, F), _row),
        pl.BlockSpec((1, BLK, F), _row),
        pl.BlockSpec((1, BLK, 1), _row),
        pl.BlockSpec((1, F, F), _wmap),
        pl.BlockSpec((1, F, F), _wmap),
        pl.BlockSpec((1, 1, F), _wmap),
    ],
    out_specs=[
        pl.BlockSpec((1, BLK, F), _row),
        pl.BlockSpec((1, BLK, 1), _row),
    ],
    out_shape=[
        jax.ShapeDtypeStruct((2, NPAD, F), _f32),
        jax.ShapeDtypeStruct((2, NPAD, 1), _f32),
    ],
)


@functools.partial(
    pl.kernel,
    out_type=(
        jax.ShapeDtypeStruct((EPAD,), _f32),
        jax.ShapeDtypeStruct((EPAD,), _f32),
    ),
    mesh=_mesh,
    scratch_types=[
        pltpu.VMEM((NPAD // 128, 128), _f32),
        pltpu.VMEM((NPAD // 128, 128), _f32),
        pltpu.VMEM((16,), _f32),
        pltpu.VMEM((EPW,), jnp.int32),
        pltpu.VMEM((EPW,), jnp.int32),
        pltpu.VMEM((EPW,), _f32),
    ],
    compiler_params=pltpu.CompilerParams(needs_layout_passes=False),
)
def _sc_predict(ad_hbm, bp_hbm, bias_hbm, es_hbm, ed_hbm, ns_hbm, nd_hbm,
                pos_out, neg_out, ad_v, bp_v, bias_v, s_v, d_v, out_v):
    c = lax.axis_index("c")
    s = lax.axis_index("s")
    w = c * NS + s
    e0 = w * EPW
    pltpu.sync_copy(ad_hbm, ad_v)
    pltpu.sync_copy(bp_hbm, bp_v)
    pltpu.sync_copy(bias_hbm, bias_v)
    bias = bias_v[...]

    def run(src_hbm, dst_hbm, out_hbm):
        pltpu.sync_copy(src_hbm.at[pl.ds(e0, EPW)], s_v)
        pltpu.sync_copy(dst_hbm.at[pl.ds(e0, EPW)], d_v)

        @pl.loop(0, EPW // 16)
        def _j(j):
            i_s = s_v[pl.ds(j * 16, 16)]
            i_d = d_v[pl.ds(j * 16, 16)]
            ga = plsc.load_gather(ad_v, [i_s >> 7, i_s & 127])
            gb = plsc.load_gather(bp_v, [i_d >> 7, i_d & 127])
            out_v[pl.ds(j * 16, 16)] = ga + gb + bias

        pltpu.sync_copy(out_v, out_hbm.at[pl.ds(e0, EPW)])

    run(es_hbm, ed_hbm, pos_out)
    run(ns_hbm, nd_hbm, neg_out)


def kernel(x_drug, x_protein, edge_src, edge_dst, neg_src, neg_dst,
           w_sd1, w_sp1, w_dp1, w_pd1, w_sd2, w_sp2, w_dp2, w_pd2,
           w_pred, b_pred):
    xd = jnp.pad(x_drug.astype(_f32), ((0, NPAD - ND), (0, 0)))
    xp = jnp.pad(x_protein.astype(_f32), ((0, NPAD - NP), (0, 0)))

    def pad_e(a):
        a = a.astype(jnp.int32)
        return jnp.concatenate([a, jnp.full((EPAD - E,), ND, jnp.int32)])

    es, ed = pad_e(edge_src), pad_e(edge_dst)
    ns2, nd2 = pad_e(neg_src), pad_e(neg_dst)

    es3 = es.reshape(NS, CPT, CH)
    ed3 = ed.reshape(NS, CPT, CH)
    zrow = jnp.zeros((NPAD, F), _f32)
    iota_r = jnp.arange(NROWS, dtype=jnp.int32)

    def layer(x_d, x_p):
        aggp, aggd, cntp, cntd = _sc_aggregate(x_d, x_p, es3, ed3, zrow, iota_r)
        x_stack = jnp.stack([x_d, x_p])
        p = jnp.stack([aggd, aggp])
        cnt = jnp.stack([cntd.reshape(NPAD, 1), cntp.reshape(NPAD, 1)])
        return x_stack, p, cnt

    x1, p1, cnt1 = layer(xd, xp)
    ws1 = jnp.stack([w_sd1, w_sp1])
    wc1 = jnp.stack([w_pd1, w_dp1])
    h1, dinv = _tc_layer1(x1, p1, cnt1, ws1, wc1)

    x2, p2, _ = layer(h1[0], h1[1])
    ws2 = jnp.stack([w_sd2, w_sp2])
    wc2 = jnp.stack([w_pd2, w_dp2])
    wp = jnp.stack([w_pred[:F, 0][None, :], w_pred[F:, 0][None, :]])
    h2, a = _tc_layer2(x2, p2, dinv, ws2, wc2, wp)

    bias16 = jnp.broadcast_to(b_pred.reshape(1).astype(_f32), (16,))
    ad2 = a[0, :, 0].reshape(NPAD // 128, 128)
    bp2 = a[1, :, 0].reshape(NPAD // 128, 128)
    pos, neg = _sc_predict(ad2, bp2, bias16, es, ed, ns2, nd2)

    return (pos[:E][:, None], neg[:E][:, None], h2[0, :ND], h2[1, :NP])


# speedup vs baseline: 7.8753x; 1.0303x over previous
import functools

import jax
import jax.numpy as jnp
from jax import lax
from jax.experimental import pallas as pl
from jax.experimental.pallas import tpu as pltpu
from jax.experimental.pallas import tpu_sc as plsc

ND = 5000
NP = 5000
E = 160000
F = 128

NPAD = 5120
EPAD = 163840
CH = 64
NC, NS = 2, 16
NW = NC * NS
CPT = EPAD // CH // NS
NB = 2
IDXB = 32
RPT = NPAD // NS
EPW = EPAD // NW
BLK = 512

_mesh = plsc.VectorSubcoreMesh(
    core_axis_name="c", subcore_axis_name="s", num_cores=NC, num_subcores=NS)

_f32 = jnp.float32


NROWS = NPAD // 128


@functools.partial(
    pl.kernel,
    out_type=(
        jax.ShapeDtypeStruct((NPAD, F), _f32),
        jax.ShapeDtypeStruct((NPAD, F), _f32),
        jax.ShapeDtypeStruct((NROWS, 128), _f32),
        jax.ShapeDtypeStruct((NROWS, 128), _f32),
    ),
    mesh=_mesh,
    scratch_types=[
        pltpu.VMEM_SHARED((NPAD, F), _f32),
        pltpu.VMEM_SHARED((NPAD, F), _f32),
        pltpu.VMEM_SHARED((NROWS, 128), _f32),
        pltpu.VMEM((IDXB, CH), jnp.int32),
        pltpu.VMEM((IDXB, CH), jnp.int32),
        pltpu.VMEM((NROWS, 128), _f32),
        pltpu.VMEM((NROWS,), jnp.int32),
        pltpu.VMEM((CH, F), _f32),
        pltpu.VMEM((CH, F), _f32),
        pltpu.SemaphoreType.DMA,
        pltpu.SemaphoreType.DMA,
        pltpu.SemaphoreType.DMA,
        pltpu.SemaphoreType.DMA,
    ],
    compiler_params=pltpu.CompilerParams(needs_layout_passes=False),
)
def _sc_aggregate(xd_hbm, xp_hbm, src3_hbm, dst3_hbm, zrow_hbm, iota_hbm,
                  aggp_out, aggd_out, cntp_out, cntd_out,
                  acc, x_sh, cnt_sh, by_idx, at_idx, hist, iota_v,
                  rows0, rows1, gsem0, gsem1, ssem0, ssem1):
    c = lax.axis_index("c")
    s = lax.axis_index("s")
    r0 = s * RPT

    pltpu.sync_copy(zrow_hbm.at[pl.ds(r0, RPT)], acc.at[pl.ds(r0, RPT)])
    pltpu.sync_copy(zrow_hbm.at[pl.ds(0, NROWS)], hist)
    pltpu.sync_copy(iota_hbm, iota_v)

    @pl.when(s == 0)
    def _():
        pltpu.sync_copy(zrow_hbm.at[pl.ds(0, NROWS)], cnt_sh)

    rows = (rows0, rows1)
    gsems = (gsem0, gsem1)
    ssems = (ssem0, ssem1)
    ones16 = jnp.ones((16,), _f32)

    def run(x_hbm, by3_hbm, at3_hbm, out_hbm, cnt_out):
        pltpu.sync_copy(x_hbm.at[pl.ds(r0, RPT)], x_sh.at[pl.ds(r0, RPT)])
        plsc.subcore_barrier()

        @pl.loop(0, CPT // IDXB)
        def _blk(q):
            pltpu.sync_copy(by3_hbm.at[s, pl.ds(q * IDXB, IDXB)], by_idx)
            pltpu.sync_copy(at3_hbm.at[s, pl.ds(q * IDXB, IDXB)], at_idx)
            for b in range(NB):
                pltpu.async_copy(x_sh.at[by_idx.at[b]], rows[b], gsems[b])

            @pl.loop(0, IDXB // NB)
            def _grp(g):
                for b in range(NB):
                    i = g * NB + b
                    pltpu.make_async_copy(x_sh.at[by_idx.at[i]], rows[b], gsems[b]).wait()
                    pltpu.async_copy(rows[b], acc.at[at_idx.at[i]], ssems[b], add=True)

                    @pl.loop(0, CH // 16)
                    def _h(j):
                        i16 = at_idx[i, pl.ds(j * 16, 16)]
                        plsc.addupdate_scatter(hist, [i16 >> 7, i16 & 127], ones16)

                @pl.when(g < IDXB // NB - 1)
                def _():
                    for b in range(NB):
                        i = g * NB + b
                        pltpu.make_async_copy(rows[b], acc.at[at_idx.at[i]], ssems[b]).wait()
                        pltpu.async_copy(x_sh.at[by_idx.at[(g + 1) * NB + b]], rows[b], gsems[b])

            last = IDXB - NB
            for b in range(NB):
                pltpu.make_async_copy(rows[b], acc.at[at_idx.at[last + b]], ssems[b]).wait()

        pltpu.sync_copy(hist, cnt_sh.at[iota_v], add=True)
        plsc.subcore_barrier()
        pltpu.sync_copy(acc.at[pl.ds(r0, RPT)], out_hbm.at[pl.ds(r0, RPT)])

        @pl.when(s == 0)
        def _():
            pltpu.sync_copy(cnt_sh, cnt_out)

    @pl.when(c == 0)
    def _():
        run(xd_hbm, src3_hbm, dst3_hbm, aggp_out, cntp_out)

    @pl.when(c == 1)
    def _():
        run(xp_hbm, dst3_hbm, src3_hbm, aggd_out, cntd_out)


_row = lambda r: (r, 0)
_wmap = lambda r: (0, 0)


def _tc_layer1_body(x_ref, p_ref, c_ref, ws_ref, wc_ref,
                    h_ref, dinv_ref):
    dinv = 1.0 / jnp.maximum(c_ref[...], 1.0)
    agg = p_ref[...] * dinv
    h = jnp.dot(x_ref[...], ws_ref[...], preferred_element_type=_f32)
    h = h + jnp.dot(agg, wc_ref[...], preferred_element_type=_f32)
    h_ref[...] = jnp.maximum(h, 0.0)
    dinv_ref[...] = dinv


_tc_layer1 = pl.pallas_call(
    _tc_layer1_body,
    grid=(NPAD // BLK,),
    in_specs=[
        pl.BlockSpec((BLK, F), _row),
        pl.BlockSpec((BLK, F), _row),
        pl.BlockSpec((BLK, 1), _row),
        pl.BlockSpec((F, F), _wmap),
        pl.BlockSpec((F, F), _wmap),
    ],
    out_specs=[
        pl.BlockSpec((BLK, F), _row),
        pl.BlockSpec((BLK, 1), _row),
    ],
    out_shape=[
        jax.ShapeDtypeStruct((NPAD, F), _f32),
        jax.ShapeDtypeStruct((NPAD, 1), _f32),
    ],
)


def _tc_layer2_body(x_ref, p_ref, dinv_ref, ws_ref, wc_ref, wp_ref,
                    h_ref, a_ref):
    agg = p_ref[...] * dinv_ref[...]
    h = jnp.dot(x_ref[...], ws_ref[...], preferred_element_type=_f32)
    h = h + jnp.dot(agg, wc_ref[...], preferred_element_type=_f32)
    h_ref[...] = h
    a_ref[...] = jnp.sum(h * wp_ref[...], axis=1, keepdims=True)


_tc_layer2 = pl.pallas_call(
    _tc_layer2_body,
    grid=(NPAD // BLK,),
    in_specs=[
        pl.BlockSpec((BLK, F), _row),
        pl.BlockSpec((BLK, F), _row),
        pl.BlockSpec((BLK, 1), _row),
        pl.BlockSpec((F, F), _wmap),
        pl.BlockSpec((F, F), _wmap),
        pl.BlockSpec((1, F), _wmap),
    ],
    out_specs=[
        pl.BlockSpec((BLK, F), _row),
        pl.BlockSpec((BLK, 1), _row),
    ],
    out_shape=[
        jax.ShapeDtypeStruct((NPAD, F), _f32),
        jax.ShapeDtypeStruct((NPAD, 1), _f32),
    ],
)


@functools.partial(
    pl.kernel,
    out_type=(
        jax.ShapeDtypeStruct((EPAD,), _f32),
        jax.ShapeDtypeStruct((EPAD,), _f32),
    ),
    mesh=_mesh,
    scratch_types=[
        pltpu.VMEM((NPAD // 128, 128), _f32),
        pltpu.VMEM((NPAD // 128, 128), _f32),
        pltpu.VMEM((16,), _f32),
        pltpu.VMEM((EPW,), jnp.int32),
        pltpu.VMEM((EPW,), jnp.int32),
        pltpu.VMEM((EPW,), _f32),
    ],
    compiler_params=pltpu.CompilerParams(needs_layout_passes=False),
)
def _sc_predict(ad_hbm, bp_hbm, bias_hbm, es_hbm, ed_hbm, ns_hbm, nd_hbm,
                pos_out, neg_out, ad_v, bp_v, bias_v, s_v, d_v, out_v):
    c = lax.axis_index("c")
    s = lax.axis_index("s")
    w = c * NS + s
    e0 = w * EPW
    pltpu.sync_copy(ad_hbm, ad_v)
    pltpu.sync_copy(bp_hbm, bp_v)
    pltpu.sync_copy(bias_hbm, bias_v)
    bias = bias_v[...]

    def run(src_hbm, dst_hbm, out_hbm):
        pltpu.sync_copy(src_hbm.at[pl.ds(e0, EPW)], s_v)
        pltpu.sync_copy(dst_hbm.at[pl.ds(e0, EPW)], d_v)

        @pl.loop(0, EPW // 16)
        def _j(j):
            i_s = s_v[pl.ds(j * 16, 16)]
            i_d = d_v[pl.ds(j * 16, 16)]
            ga = plsc.load_gather(ad_v, [i_s >> 7, i_s & 127])
            gb = plsc.load_gather(bp_v, [i_d >> 7, i_d & 127])
            out_v[pl.ds(j * 16, 16)] = ga + gb + bias

        pltpu.sync_copy(out_v, out_hbm.at[pl.ds(e0, EPW)])

    run(es_hbm, ed_hbm, pos_out)
    run(ns_hbm, nd_hbm, neg_out)


def kernel(x_drug, x_protein, edge_src, edge_dst, neg_src, neg_dst,
           w_sd1, w_sp1, w_dp1, w_pd1, w_sd2, w_sp2, w_dp2, w_pd2,
           w_pred, b_pred):
    xd = jnp.pad(x_drug.astype(_f32), ((0, NPAD - ND), (0, 0)))
    xp = jnp.pad(x_protein.astype(_f32), ((0, NPAD - NP), (0, 0)))

    def pad_e(a):
        a = a.astype(jnp.int32)
        return jnp.concatenate([a, jnp.full((EPAD - E,), ND, jnp.int32)])

    es, ed = pad_e(edge_src), pad_e(edge_dst)
    ns2, nd2 = pad_e(neg_src), pad_e(neg_dst)

    es3 = es.reshape(NS, CPT, CH)
    ed3 = ed.reshape(NS, CPT, CH)
    zrow = jnp.zeros((NPAD, F), _f32)
    iota_r = jnp.arange(NROWS, dtype=jnp.int32)

    aggp1, aggd1, cntp1, cntd1 = _sc_aggregate(xd, xp, es3, ed3, zrow, iota_r)
    h1d, dinv_d = _tc_layer1(xd, aggd1, cntd1.reshape(NPAD, 1), w_sd1, w_pd1)
    h1p, dinv_p = _tc_layer1(xp, aggp1, cntp1.reshape(NPAD, 1), w_sp1, w_dp1)

    aggp2, aggd2, _, _ = _sc_aggregate(h1d, h1p, es3, ed3, zrow, iota_r)
    wp_d = w_pred[:F, 0][None, :]
    wp_p = w_pred[F:, 0][None, :]
    h2d, a_d = _tc_layer2(h1d, aggd2, dinv_d, w_sd2, w_pd2, wp_d)
    h2p, b_p = _tc_layer2(h1p, aggp2, dinv_p, w_sp2, w_dp2, wp_p)

    bias16 = jnp.broadcast_to(b_pred.reshape(1).astype(_f32), (16,))
    ad2 = a_d.reshape(NPAD // 128, 128)
    bp2 = b_p.reshape(NPAD // 128, 128)
    pos, neg = _sc_predict(ad2, bp2, bias16, es, ed, ns2, nd2)

    return (pos[:E][:, None], neg[:E][:, None], h2d[:ND], h2p[:NP])
